# baseline (speedup 1.0000x reference)
import functools

import jax
import jax.numpy as jnp
from jax import lax
from jax.experimental import pallas as pl
from jax.experimental.pallas import tpu as pltpu
from jax.experimental.pallas import tpu_sc as plsc

_NUM_EMB = 1000000
_EMB_DIM = 64
_GROUP_SIZE = 32

_NW = 32
_CHUNK = 128
_LANES = 16


def _sc_gather(weight, szp8, idx1d, n_chunks_per_worker):
    ncw = n_chunks_per_worker
    per_w = ncw * _CHUNK
    b_total = _NW * per_w
    mesh = plsc.VectorSubcoreMesh(core_axis_name="c", subcore_axis_name="s")

    @functools.partial(
        pl.kernel,
        out_type=[
            jax.ShapeDtypeStruct((b_total, _EMB_DIM), jnp.int8),
            jax.ShapeDtypeStruct((b_total, 8), jnp.float32),
        ],
        mesh=mesh,
        compiler_params=pltpu.CompilerParams(use_tc_tiling_on_sc=False,
                                             needs_layout_passes=False),
        scratch_types=[
            pltpu.VMEM((per_w,), jnp.int32),
            pltpu.VMEM((_CHUNK,), jnp.int32),
            pltpu.VMEM((_CHUNK, _EMB_DIM), jnp.int8),
            pltpu.VMEM((_CHUNK, 8), jnp.float32),
            pltpu.VMEM((_CHUNK, 8), jnp.float32),
            pltpu.SemaphoreType.DMA,
            pltpu.SemaphoreType.DMA,
        ],
    )
    def gather_kernel(w_hbm, szp_hbm, idx_hbm, wg_hbm, szpg_hbm,
                      idx_v, hidx_v, w_v, t_v, c_v, sem_w, sem_s):
        wid = lax.axis_index("s") * 2 + lax.axis_index("c")
        base = wid * per_w
        pltpu.sync_copy(idx_hbm.at[pl.ds(base, per_w)], idx_v)

        def body(g, carry):
            idx_c = idx_v.at[pl.ds(g * _CHUNK, _CHUNK)]
            cw = pltpu.async_copy(w_hbm.at[idx_c], w_v, sem_w)
            for k in range(_CHUNK // _LANES):
                iv = idx_v[pl.ds(g * _CHUNK + k * _LANES, _LANES)]
                hidx_v[pl.ds(k * _LANES, _LANES)] = iv >> 1
            cs = pltpu.async_copy(szp_hbm.at[hidx_v], t_v, sem_s)
            cs.wait()
            for k in range(_CHUNK // _LANES):
                iv = idx_v[pl.ds(g * _CHUNK + k * _LANES, _LANES)]
                o = (iv & 1) * 4
                r = lax.iota(jnp.int32, _LANES) + k * _LANES
                zero = jnp.zeros((_LANES,), jnp.int32)
                for c in range(4):
                    v = plsc.load_gather(t_v, [r, o + c])
                    plsc.store_scatter(c_v, [r, zero + c], v)
            cw.wait()
            row0 = base + g * _CHUNK
            pltpu.sync_copy(w_v, wg_hbm.at[pl.ds(row0, _CHUNK)])
            pltpu.sync_copy(c_v, szpg_hbm.at[pl.ds(row0, _CHUNK)])
            return carry

        lax.fori_loop(0, ncw, body, 0)

    return gather_kernel(weight, szp8, idx1d)


def _tc_dequant(w8, szp_g, blk):
    b_total = w8.shape[0]

    def body(w_ref, szp_ref, o_ref):
        w = w_ref[...].astype(jnp.float32)
        szp = szp_ref[...]
        s_full = jnp.concatenate(
            [jnp.broadcast_to(szp[:, 0:1], (blk, _GROUP_SIZE)),
             jnp.broadcast_to(szp[:, 1:2], (blk, _GROUP_SIZE))], axis=1)
        z_full = jnp.concatenate(
            [jnp.broadcast_to(szp[:, 2:3], (blk, _GROUP_SIZE)),
             jnp.broadcast_to(szp[:, 3:4], (blk, _GROUP_SIZE))], axis=1)
        o_ref[...] = (w - z_full) * s_full

    return pl.pallas_call(
        body,
        grid=(b_total // blk,),
        in_specs=[
            pl.BlockSpec((blk, _EMB_DIM), lambda i: (i, 0)),
            pl.BlockSpec((blk, 8), lambda i: (i, 0)),
        ],
        out_specs=pl.BlockSpec((blk, _EMB_DIM), lambda i: (i, 0)),
        out_shape=jax.ShapeDtypeStruct((b_total, _EMB_DIM), jnp.float32),
    )(w8, szp_g)


def kernel(x, weight, scale, zero_point):
    b, s = x.shape
    n_idx = b * s
    ncw = n_idx // (_NW * _CHUNK)
    idx1d = x.reshape(n_idx)

    szp8 = jnp.concatenate([scale, zero_point], axis=1).reshape(_NUM_EMB // 2, 8)

    w_g, szp_g = _sc_gather(weight, szp8, idx1d, ncw)
    out = _tc_dequant(w_g, szp_g, blk=2048)
    return out.reshape(b, s, _EMB_DIM)


# speedup vs baseline: 1.0462x; 1.0462x over previous
import functools

import jax
import jax.numpy as jnp
from jax import lax
from jax.experimental import pallas as pl
from jax.experimental.pallas import tpu as pltpu
from jax.experimental.pallas import tpu_sc as plsc

_NUM_EMB = 1000000
_EMB_DIM = 64
_GROUP_SIZE = 32

_NW = 32
_CHUNK = 128
_LANES = 16


def _sc_gather(weight, szp8, idx1d, n_chunks_per_worker):
    ncw = n_chunks_per_worker
    per_w = ncw * _CHUNK
    b_total = _NW * per_w
    mesh = plsc.VectorSubcoreMesh(core_axis_name="c", subcore_axis_name="s")

    @functools.partial(
        pl.kernel,
        out_type=[
            jax.ShapeDtypeStruct((b_total, _EMB_DIM), jnp.int8),
            jax.ShapeDtypeStruct((b_total, 8), jnp.float32),
        ],
        mesh=mesh,
        compiler_params=pltpu.CompilerParams(use_tc_tiling_on_sc=False,
                                             needs_layout_passes=False),
        scratch_types=[
            pltpu.VMEM((per_w,), jnp.int32),
            pltpu.VMEM((_CHUNK,), jnp.int32),
            pltpu.VMEM((_CHUNK, _EMB_DIM), jnp.int8),
            pltpu.VMEM((_CHUNK, 8), jnp.float32),
            pltpu.VMEM((_CHUNK, 8), jnp.float32),
            pltpu.SemaphoreType.DMA,
            pltpu.SemaphoreType.DMA,
        ],
    )
    def gather_kernel(w_hbm, szp_hbm, idx_hbm, wg_hbm, szpg_hbm,
                      idx_v, hidx_v, w_v, t_v, c_v, sem_w, sem_s):
        wid = lax.axis_index("s") * 2 + lax.axis_index("c")
        base = wid * per_w
        pltpu.sync_copy(idx_hbm.at[pl.ds(base, per_w)], idx_v)

        def body(g, carry):
            idx_c = idx_v.at[pl.ds(g * _CHUNK, _CHUNK)]
            cw = pltpu.async_copy(w_hbm.at[idx_c], w_v, sem_w)
            for k in range(_CHUNK // _LANES):
                iv = idx_v[pl.ds(g * _CHUNK + k * _LANES, _LANES)]
                hidx_v[pl.ds(k * _LANES, _LANES)] = iv >> 1
            cs = pltpu.async_copy(szp_hbm.at[hidx_v], t_v, sem_s)
            cs.wait()
            for k in range(_CHUNK // _LANES):
                iv = idx_v[pl.ds(g * _CHUNK + k * _LANES, _LANES)]
                o = (iv & 1) * 4
                r = lax.iota(jnp.int32, _LANES) + k * _LANES
                zero = jnp.zeros((_LANES,), jnp.int32)
                for c in range(4):
                    v = plsc.load_gather(t_v, [r, o + c])
                    plsc.store_scatter(c_v, [r, zero + c], v)
            cw.wait()
            row0 = base + g * _CHUNK
            pltpu.sync_copy(w_v, wg_hbm.at[pl.ds(row0, _CHUNK)])
            pltpu.sync_copy(c_v, szpg_hbm.at[pl.ds(row0, _CHUNK)])
            return carry

        lax.fori_loop(0, ncw, body, 0)

    return gather_kernel(weight, szp8, idx1d)


def _tc_row_major(weight):
    blk = 8192
    wT = weight.T

    def body(wt_ref, o_ref):
        t = wt_ref[...].astype(jnp.int32)
        o_ref[...] = t.T.astype(jnp.int8)

    return pl.pallas_call(
        body,
        grid=(pl.cdiv(_NUM_EMB, blk),),
        in_specs=[pl.BlockSpec((_EMB_DIM, blk), lambda i: (0, i))],
        out_specs=pl.BlockSpec((blk, _EMB_DIM), lambda i: (i, 0)),
        out_shape=jax.ShapeDtypeStruct((_NUM_EMB, _EMB_DIM), jnp.int8),
    )(wT)


def _tc_dequant(w8, szp_g, blk):
    b_total = w8.shape[0]

    def body(w_ref, szp_ref, o_ref):
        w = w_ref[...].astype(jnp.float32)
        szp = szp_ref[...]
        s_full = jnp.concatenate(
            [jnp.broadcast_to(szp[:, 0:1], (blk, _GROUP_SIZE)),
             jnp.broadcast_to(szp[:, 1:2], (blk, _GROUP_SIZE))], axis=1)
        z_full = jnp.concatenate(
            [jnp.broadcast_to(szp[:, 2:3], (blk, _GROUP_SIZE)),
             jnp.broadcast_to(szp[:, 3:4], (blk, _GROUP_SIZE))], axis=1)
        o_ref[...] = (w - z_full) * s_full

    return pl.pallas_call(
        body,
        grid=(b_total // blk,),
        in_specs=[
            pl.BlockSpec((blk, _EMB_DIM), lambda i: (i, 0)),
            pl.BlockSpec((blk, 8), lambda i: (i, 0)),
        ],
        out_specs=pl.BlockSpec((blk, _EMB_DIM), lambda i: (i, 0)),
        out_shape=jax.ShapeDtypeStruct((b_total, _EMB_DIM), jnp.float32),
    )(w8, szp_g)


def kernel(x, weight, scale, zero_point):
    b, s = x.shape
    n_idx = b * s
    ncw = n_idx // (_NW * _CHUNK)
    idx1d = x.reshape(n_idx)

    szp8 = jnp.concatenate([scale, zero_point], axis=1).reshape(_NUM_EMB // 2, 8)

    w_rm = _tc_row_major(weight)
    w_g, szp_g = _sc_gather(w_rm, szp8, idx1d, ncw)
    out = _tc_dequant(w_g, szp_g, blk=2048)
    return out.reshape(b, s, _EMB_DIM)


# speedup vs baseline: 1.1373x; 1.0872x over previous
import functools

import jax
import jax.numpy as jnp
from jax import lax
from jax.experimental import pallas as pl
from jax.experimental.pallas import tpu as pltpu
from jax.experimental.pallas import tpu_sc as plsc

_NUM_EMB = 1000000
_EMB_DIM = 64
_GROUP_SIZE = 32

_NW = 32
_CHUNK = 128
_LANES = 16


def _sc_gather(weight, szp8, idx1d, n_chunks_per_worker):
    ncw = n_chunks_per_worker
    per_w = ncw * _CHUNK
    b_total = _NW * per_w
    mesh = plsc.VectorSubcoreMesh(core_axis_name="c", subcore_axis_name="s")

    @functools.partial(
        pl.kernel,
        out_type=[
            jax.ShapeDtypeStruct((b_total, _EMB_DIM), jnp.int8),
            jax.ShapeDtypeStruct((b_total, 8), jnp.float32),
        ],
        mesh=mesh,
        compiler_params=pltpu.CompilerParams(use_tc_tiling_on_sc=False,
                                             needs_layout_passes=False),
        scratch_types=[
            pltpu.VMEM((per_w,), jnp.int32),
            pltpu.VMEM((_CHUNK,), jnp.int32),
            pltpu.VMEM((_CHUNK, _EMB_DIM), jnp.int8),
            pltpu.VMEM((_CHUNK, 8), jnp.float32),
            pltpu.VMEM((_CHUNK, 8), jnp.float32),
            pltpu.SemaphoreType.DMA,
            pltpu.SemaphoreType.DMA,
        ],
    )
    def gather_kernel(w_hbm, szp_hbm, idx_hbm, wg_hbm, szpg_hbm,
                      idx_v, hidx_v, w_v, t_v, c_v, sem_w, sem_s):
        wid = lax.axis_index("s") * 2 + lax.axis_index("c")
        base = wid * per_w
        pltpu.sync_copy(idx_hbm.at[pl.ds(base, per_w)], idx_v)

        def body(g, carry):
            idx_c = idx_v.at[pl.ds(g * _CHUNK, _CHUNK)]
            cw = pltpu.async_copy(w_hbm.at[idx_c], w_v, sem_w)
            for k in range(_CHUNK // _LANES):
                iv = idx_v[pl.ds(g * _CHUNK + k * _LANES, _LANES)]
                hidx_v[pl.ds(k * _LANES, _LANES)] = iv >> 1
            cs = pltpu.async_copy(szp_hbm.at[hidx_v], t_v, sem_s)
            cs.wait()
            for k in range(_CHUNK // _LANES):
                iv = idx_v[pl.ds(g * _CHUNK + k * _LANES, _LANES)]
                o = (iv & 1) * 4
                r = lax.iota(jnp.int32, _LANES) + k * _LANES
                zero = jnp.zeros((_LANES,), jnp.int32)
                for c in range(4):
                    v = plsc.load_gather(t_v, [r, o + c])
                    plsc.store_scatter(c_v, [r, zero + c], v)
            cw.wait()
            row0 = base + g * _CHUNK
            pltpu.sync_copy(w_v, wg_hbm.at[pl.ds(row0, _CHUNK)])
            pltpu.sync_copy(c_v, szpg_hbm.at[pl.ds(row0, _CHUNK)])
            return carry

        lax.fori_loop(0, ncw, body, 0)

    return gather_kernel(weight, szp8, idx1d)


def _tc_row_major(weight):
    blk = 8192
    wT = weight.T

    def body(wt_ref, o_ref):
        t = wt_ref[...].astype(jnp.int32)
        o_ref[...] = t.T.astype(jnp.int8)

    return pl.pallas_call(
        body,
        grid=(pl.cdiv(_NUM_EMB, blk),),
        in_specs=[pl.BlockSpec((_EMB_DIM, blk), lambda i: (0, i))],
        out_specs=pl.BlockSpec((blk, _EMB_DIM), lambda i: (i, 0)),
        out_shape=jax.ShapeDtypeStruct((_NUM_EMB, _EMB_DIM), jnp.int8),
    )(wT)


def _tc_dequant_t(w8, szp_g, n_s, n_b):
    def body(w_ref, szp_ref, o_ref):
        w = w_ref[...].astype(jnp.float32)
        szp = szp_ref[...]
        s_full = jnp.concatenate(
            [jnp.broadcast_to(szp[:, 0:1], (n_b, _GROUP_SIZE)),
             jnp.broadcast_to(szp[:, 1:2], (n_b, _GROUP_SIZE))], axis=1)
        z_full = jnp.concatenate(
            [jnp.broadcast_to(szp[:, 2:3], (n_b, _GROUP_SIZE)),
             jnp.broadcast_to(szp[:, 3:4], (n_b, _GROUP_SIZE))], axis=1)
        o_ref[0] = ((w - z_full) * s_full).T

    return pl.pallas_call(
        body,
        grid=(n_s,),
        in_specs=[
            pl.BlockSpec((n_b, _EMB_DIM), lambda i: (i, 0)),
            pl.BlockSpec((n_b, 8), lambda i: (i, 0)),
        ],
        out_specs=pl.BlockSpec((1, _EMB_DIM, n_b), lambda i: (i, 0, 0)),
        out_shape=jax.ShapeDtypeStruct((n_s, _EMB_DIM, n_b), jnp.float32),
    )(w8, szp_g)


def kernel(x, weight, scale, zero_point):
    b, s = x.shape
    n_idx = b * s
    ncw = n_idx // (_NW * _CHUNK)
    idx1d = x.T.reshape(n_idx)

    szp8 = jnp.concatenate([scale, zero_point], axis=1).reshape(_NUM_EMB // 2, 8)

    w_rm = _tc_row_major(weight)
    w_g, szp_g = _sc_gather(w_rm, szp8, idx1d, ncw)
    out3 = _tc_dequant_t(w_g, szp_g, s, b)
    return jnp.transpose(out3, (2, 0, 1))
